# baseline (speedup 1.0000x reference)
import functools

import jax
import jax.numpy as jnp
from jax import lax
from jax.experimental import pallas as pl
from jax.experimental.pallas import tpu as pltpu
from jax.experimental.pallas import tpu_sc as plsc

_B, _T, _D = 4, 2048, 768
_NC, _NS, _L = 2, 16, 16
_NW = _NC * _NS
_PB = _T // _NW
_DV = _D // _L

_mesh = plsc.VectorSubcoreMesh(core_axis_name="c", subcore_axis_name="s")


@functools.partial(
    pl.kernel,
    out_type=jax.ShapeDtypeStruct((_B, _T, _D), jnp.float32),
    mesh=_mesh,
    scratch_types=[
        pltpu.VMEM((_PB,), jnp.int32),
        pltpu.VMEM((_PB, _D), jnp.float32),
        pltpu.VMEM((_PB, _D), jnp.float32),
        pltpu.SemaphoreType.DMA,
    ],
)
def _emb_kernel(idx_hbm, tok_hbm, pos_hbm, out_hbm, idx_v, pos_v, rows_v, sem):
    wid = lax.axis_index("s") * _NC + lax.axis_index("c")
    p0 = wid * _PB
    pltpu.sync_copy(pos_hbm.at[0, pl.ds(p0, _PB)], pos_v)
    for b in range(_B):
        pltpu.sync_copy(idx_hbm.at[b, pl.ds(p0, _PB)], idx_v)
        pltpu.async_copy(tok_hbm.at[idx_v], rows_v, sem).wait()

        def row_body(i, _):
            def lane_body(j, _):
                sl = pl.ds(j * _L, _L)
                rows_v[i, sl] = rows_v[i, sl] + pos_v[i, sl]
                return 0

            return lax.fori_loop(0, _DV, lane_body, 0)

        lax.fori_loop(0, _PB, row_body, 0)
        pltpu.sync_copy(rows_v, out_hbm.at[b, pl.ds(p0, _PB)])


def kernel(idx, tok_emb, pos_emb):
    return _emb_kernel(idx.astype(jnp.int32), tok_emb, pos_emb)


# speedup vs baseline: 1.3805x; 1.3805x over previous
import functools

import jax
import jax.numpy as jnp
from jax import lax
from jax.experimental import pallas as pl
from jax.experimental.pallas import tpu as pltpu
from jax.experimental.pallas import tpu_sc as plsc

_B, _T, _D = 4, 2048, 768
_NC, _NS, _L = 2, 16, 16
_NW = _NC * _NS
_PB = _T // _NW
_CR = 32
_NCH = _B * _PB // _CR
_DV = _D // _L

_mesh = plsc.VectorSubcoreMesh(core_axis_name="c", subcore_axis_name="s")


@functools.partial(
    pl.kernel,
    out_type=jax.ShapeDtypeStruct((_B, _T, _D), jnp.float32),
    mesh=_mesh,
    scratch_types=[
        pltpu.VMEM((_B * _PB,), jnp.int32),
        pltpu.VMEM((_PB, _D), jnp.float32),
        pltpu.VMEM((_CR, _D), jnp.float32),
        pltpu.VMEM((_CR, _D), jnp.float32),
        pltpu.SemaphoreType.DMA,
        pltpu.SemaphoreType.DMA,
        pltpu.SemaphoreType.DMA,
        pltpu.SemaphoreType.DMA,
    ],
)
def _emb_kernel(idx_hbm, tok_hbm, pos_hbm, out_hbm,
                idx_v, pos_v, buf0, buf1, sg0, sg1, ss0, ss1):
    wid = lax.axis_index("s") * _NC + lax.axis_index("c")
    p0 = wid * _PB
    bufs, gsem, ssem = [buf0, buf1], [sg0, sg1], [ss0, ss1]

    for b in range(_B):
        pltpu.sync_copy(idx_hbm.at[b, pl.ds(p0, _PB)],
                        idx_v.at[pl.ds(b * _PB, _PB)])
    pltpu.sync_copy(pos_hbm.at[0, pl.ds(p0, _PB)], pos_v)

    gather_h = [None] * _NCH
    store_h = [None] * _NCH

    def start_gather(c):
        b, h = divmod(c, _PB // _CR)
        gather_h[c] = pltpu.async_copy(
            tok_hbm.at[idx_v.at[pl.ds(b * _PB + h * _CR, _CR)]],
            bufs[c % 2], gsem[c % 2])

    start_gather(0)
    for c in range(_NCH):
        if c + 1 < _NCH:
            if c >= 1:
                store_h[c - 1].wait()
            start_gather(c + 1)
        gather_h[c].wait()
        b, h = divmod(c, _PB // _CR)
        buf = bufs[c % 2]

        def row_body(i, _, buf=buf, h=h):
            for j in range(_DV):
                sl = pl.ds(j * _L, _L)
                buf[i, sl] = buf[i, sl] + pos_v[h * _CR + i, sl]
            return 0

        lax.fori_loop(0, _CR, row_body, 0)
        store_h[c] = pltpu.async_copy(
            buf, out_hbm.at[b, pl.ds(p0 + h * _CR, _CR)], ssem[c % 2])
    store_h[_NCH - 2].wait()
    store_h[_NCH - 1].wait()


def kernel(idx, tok_emb, pos_emb):
    return _emb_kernel(idx.astype(jnp.int32), tok_emb, pos_emb)


# speedup vs baseline: 1.6468x; 1.1929x over previous
import functools

import jax
import jax.numpy as jnp
from jax import lax
from jax.experimental import pallas as pl
from jax.experimental.pallas import tpu as pltpu
from jax.experimental.pallas import tpu_sc as plsc

_B, _T, _D = 4, 2048, 768
_NC, _NS, _L = 2, 16, 16
_NW = _NC * _NS
_PB = _T // _NW
_CR = 16
_NCH = _B * _PB // _CR
_NBUF = 4
_DV = _D // _L

_mesh = plsc.VectorSubcoreMesh(core_axis_name="c", subcore_axis_name="s")


@functools.partial(
    pl.kernel,
    out_type=jax.ShapeDtypeStruct((_B, _T, _D), jnp.float32),
    mesh=_mesh,
    scratch_types=[
        pltpu.VMEM((_B * _PB,), jnp.int32),
        pltpu.VMEM((_PB, _D), jnp.float32),
    ]
    + [pltpu.VMEM((_CR, _D), jnp.float32) for _ in range(_NBUF)]
    + [pltpu.SemaphoreType.DMA for _ in range(2 * _NBUF)],
)
def _emb_kernel(idx_hbm, tok_hbm, pos_hbm, out_hbm, idx_v, pos_v, *rest):
    bufs = rest[:_NBUF]
    gsem = rest[_NBUF:2 * _NBUF]
    ssem = rest[2 * _NBUF:]
    wid = lax.axis_index("s") * _NC + lax.axis_index("c")
    p0 = wid * _PB

    for b in range(_B):
        pltpu.sync_copy(idx_hbm.at[b, pl.ds(p0, _PB)],
                        idx_v.at[pl.ds(b * _PB, _PB)])
    pos_h = pltpu.async_copy(pos_hbm.at[0, pl.ds(p0, _PB)], pos_v, ssem[0])

    gather_h = [None] * _NCH
    store_h = [None] * _NCH
    hpb = _PB // _CR

    def start_gather(c):
        b, h = divmod(c, hpb)
        gather_h[c] = pltpu.async_copy(
            tok_hbm.at[idx_v.at[pl.ds(b * _PB + h * _CR, _CR)]],
            bufs[c % _NBUF], gsem[c % _NBUF])

    for c in range(_NBUF - 1):
        start_gather(c)
    pos_h.wait()

    for c in range(_NCH):
        nxt = c + _NBUF - 1
        if nxt < _NCH:
            if c >= 1:
                store_h[c - 1].wait()
            start_gather(nxt)
        gather_h[c].wait()
        b, h = divmod(c, hpb)
        buf = bufs[c % _NBUF]

        @plsc.parallel_loop(0, _CR)
        def _add(i, buf=buf, h=h):
            for j in range(_DV):
                sl = pl.ds(j * _L, _L)
                buf[i, sl] = buf[i, sl] + pos_v[h * _CR + i, sl]

        store_h[c] = pltpu.async_copy(
            buf, out_hbm.at[b, pl.ds(p0 + h * _CR, _CR)], ssem[c % _NBUF])
    for c in range(_NCH - _NBUF, _NCH):
        store_h[c].wait()


def kernel(idx, tok_emb, pos_emb):
    return _emb_kernel(idx.astype(jnp.int32), tok_emb, pos_emb)


# speedup vs baseline: 1.8907x; 1.1481x over previous
import functools

import jax
import jax.numpy as jnp
from jax import lax
from jax.experimental import pallas as pl
from jax.experimental.pallas import tpu as pltpu
from jax.experimental.pallas import tpu_sc as plsc

_B, _T, _D = 4, 2048, 768
_NC, _NS, _L = 2, 16, 16
_NW = _NC * _NS
_PB = _T // _NW
_CR = 8
_NG = _PB // _CR
_NBUF = 2 * _B
_DV = _D // _L

_mesh = plsc.VectorSubcoreMesh(core_axis_name="c", subcore_axis_name="s")


@functools.partial(
    pl.kernel,
    out_type=jax.ShapeDtypeStruct((_B, _T, _D), jnp.float32),
    mesh=_mesh,
    scratch_types=[
        pltpu.VMEM((_B * _PB,), jnp.int32),
        pltpu.VMEM((_PB, _D), jnp.float32),
        pltpu.SemaphoreType.DMA,
    ]
    + [pltpu.VMEM((_CR, _D), jnp.float32) for _ in range(_NBUF)]
    + [pltpu.SemaphoreType.DMA for _ in range(2 * _NBUF)],
)
def _emb_kernel(idx_hbm, tok_hbm, pos_hbm, out_hbm, idx_v, pos_v, isem, *rest):
    bufs = rest[:_NBUF]
    gsem = rest[_NBUF:2 * _NBUF]
    ssem = rest[2 * _NBUF:]
    wid = lax.axis_index("s") * _NC + lax.axis_index("c")
    p0 = wid * _PB

    idx_h = [pltpu.async_copy(idx_hbm.at[b, pl.ds(p0, _PB)],
                              idx_v.at[pl.ds(b * _PB, _PB)], isem)
             for b in range(_B)]
    pos_h = pltpu.async_copy(pos_hbm.at[0, pl.ds(p0, _PB)], pos_v, ssem[0])

    gather_h = [[None] * _B for _ in range(_NG)]
    store_h = [[None] * _B for _ in range(_NG)]

    def start_group(g):
        bank = (g % 2) * _B
        for b in range(_B):
            gather_h[g][b] = pltpu.async_copy(
                tok_hbm.at[idx_v.at[pl.ds(b * _PB + g * _CR, _CR)]],
                bufs[bank + b], gsem[bank + b])

    for h in idx_h:
        h.wait()
    start_group(0)
    pos_h.wait()

    for g in range(_NG):
        bank = (g % 2) * _B
        if g + 1 < _NG:
            if g >= 1:
                for b in range(_B):
                    store_h[g - 1][b].wait()
            start_group(g + 1)
        for b in range(_B):
            gather_h[g][b].wait()
        gbufs = bufs[bank:bank + _B]

        @plsc.parallel_loop(0, _CR)
        def _add(i, gbufs=gbufs, g=g):
            for j in range(_DV):
                sl = pl.ds(j * _L, _L)
                vp = pos_v[g * _CR + i, sl]
                for b in range(_B):
                    gbufs[b][i, sl] = gbufs[b][i, sl] + vp

        for b in range(_B):
            store_h[g][b] = pltpu.async_copy(
                bufs[bank + b],
                out_hbm.at[b, pl.ds(p0 + g * _CR, _CR)], ssem[bank + b])
    for b in range(_B):
        store_h[_NG - 2][b].wait()
        store_h[_NG - 1][b].wait()


def kernel(idx, tok_emb, pos_emb):
    return _emb_kernel(idx.astype(jnp.int32), tok_emb, pos_emb)
